# baseline (speedup 1.0000x reference)
import jax
import jax.numpy as jnp
from jax.experimental import pallas as pl
from jax.experimental.pallas import tpu as pltpu


def _tc_body(ft_ref, c_ref, out_ref, cb_ref, csq_ref):
    i = pl.program_id(0)
    nsteps = pl.num_programs(0)

    @pl.when(i == 0)
    def _():
        c = c_ref[...]
        cb_ref[...] = (c * -2.0).astype(jnp.bfloat16)
        csq_ref[...] = jnp.sum(c * c, axis=1, keepdims=True)
        out_ref[0, 0] = 0.0

    ft = ft_ref[...]
    fsq = jnp.sum(ft * ft, axis=0, keepdims=True)
    fb = ft.astype(jnp.bfloat16)
    dot = jax.lax.dot_general(
        cb_ref[...], fb, (((1,), (0,)), ((), ())),
        preferred_element_type=jnp.float32)
    sq = dot + csq_ref[...]
    minsq = jnp.min(sq, axis=0, keepdims=True)
    dist = jnp.minimum(jnp.sqrt(jnp.maximum(minsq + fsq, 0.0)), 1000000.0)
    out_ref[0, 0] += jnp.sum(dist)

    @pl.when(i == nsteps - 1)
    def _():
        out_ref[0, 0] = out_ref[0, 0] * (1.0 / (nsteps * ft.shape[1]))


def kernel(features, centers):
    n, d = features.shape
    k = centers.shape[0]
    bn = 2048
    ft = features.T

    out = pl.pallas_call(
        _tc_body,
        grid=(n // bn,),
        in_specs=[
            pl.BlockSpec((d, bn), lambda i: (0, i)),
            pl.BlockSpec((k, d), lambda i: (0, 0)),
        ],
        out_specs=pl.BlockSpec((1, 1), lambda i: (0, 0),
                               memory_space=pltpu.SMEM),
        out_shape=jax.ShapeDtypeStruct((1, 1), jnp.float32),
        scratch_shapes=[pltpu.VMEM((k, d), jnp.bfloat16),
                        pltpu.VMEM((k, 1), jnp.float32)],
    )(ft, centers)
    return out[0, 0]


# speedup vs baseline: 1.0164x; 1.0164x over previous
import jax
import jax.numpy as jnp
from jax.experimental import pallas as pl
from jax.experimental.pallas import tpu as pltpu


def _tc_body(ft_ref, c_ref, out_ref, cb_ref):
    i = pl.program_id(0)
    nsteps = pl.num_programs(0)

    @pl.when(i == 0)
    def _():
        c = c_ref[...]
        csq = jnp.sum(c * c, axis=1, keepdims=True)
        csq_hi = csq.astype(jnp.bfloat16)
        csq_lo = (csq - csq_hi.astype(jnp.float32)).astype(jnp.bfloat16)
        cb_ref[...] = jnp.concatenate(
            [(c * -2.0).astype(jnp.bfloat16), csq_hi, csq_lo], axis=1)
        out_ref[0, 0] = 0.0

    ft = ft_ref[...]
    fsq = jnp.sum(ft * ft, axis=0, keepdims=True)
    fb = ft.astype(jnp.bfloat16)
    ones2 = jnp.ones((2, ft.shape[1]), jnp.bfloat16)
    faug = jnp.concatenate([fb, ones2], axis=0)
    sq = jax.lax.dot_general(
        cb_ref[...], faug, (((1,), (0,)), ((), ())),
        preferred_element_type=jnp.float32)
    minsq = jnp.min(sq, axis=0, keepdims=True)
    dist = jnp.minimum(jnp.sqrt(jnp.maximum(minsq + fsq, 0.0)), 1000000.0)
    out_ref[0, 0] += jnp.sum(dist)

    @pl.when(i == nsteps - 1)
    def _():
        out_ref[0, 0] = out_ref[0, 0] * (1.0 / (nsteps * ft.shape[1]))


def kernel(features, centers):
    n, d = features.shape
    k = centers.shape[0]
    bn = 2048
    ft = features.T

    out = pl.pallas_call(
        _tc_body,
        grid=(n // bn,),
        in_specs=[
            pl.BlockSpec((d, bn), lambda i: (0, i)),
            pl.BlockSpec((k, d), lambda i: (0, 0)),
        ],
        out_specs=pl.BlockSpec((1, 1), lambda i: (0, 0),
                               memory_space=pltpu.SMEM),
        out_shape=jax.ShapeDtypeStruct((1, 1), jnp.float32),
        scratch_shapes=[pltpu.VMEM((k, d + 2), jnp.bfloat16)],
    )(ft, centers)
    return out[0, 0]
